# Initial kernel scaffold; baseline (speedup 1.0000x reference)
import jax
import jax.numpy as jnp
from jax.experimental import pallas as pl


def kernel(x, ln_w, ln_b, gate_w, W1, W2):
    raise NotImplementedError("write your pallas kernel here")


# speedup vs baseline: 2.7747x; 2.7747x over previous
import functools

import jax
import jax.numpy as jnp
from jax.experimental import pallas as pl
from jax.experimental.pallas import tpu as pltpu

E = 8
TOP_K = 2
D = 1024
FF = 2048
FFC = 512
NFF = FF // FFC


def _gelu_tanh(h):
    c = 0.7978845608028654
    return 0.5 * h * (1.0 + jnp.tanh(c * (h + 0.044715 * h * h * h)))


def _moe_body(x_ref, lnw_ref, lnb_ref, gate_ref, w1_ref, w2_ref,
              out_ref, coef_ref):
    e = pl.program_id(0)
    f = pl.program_id(1)
    step = e * NFF + f

    @pl.when(step == 0)
    def _router():
        xf = x_ref[...]
        mu = jnp.mean(xf, axis=1, keepdims=True)
        xc = xf - mu
        var = jnp.mean(xc * xc, axis=1, keepdims=True)
        ri = xc * jax.lax.rsqrt(var + 1e-5) * lnw_ref[...][None, :] \
            + lnb_ref[...][None, :]
        logits = jax.lax.dot_general(
            ri, gate_ref[...], (((1,), (1,)), ((), ())),
            preferred_element_type=jnp.float32)
        mx = jnp.max(logits, axis=1, keepdims=True)
        p = jnp.exp(logits - mx)
        p = p / jnp.sum(p, axis=1, keepdims=True)
        m1 = jnp.max(p, axis=1, keepdims=True)
        is1 = p >= m1
        p2 = jnp.where(is1, -jnp.inf, p)
        m2 = jnp.max(p2, axis=1, keepdims=True)
        is2 = p2 >= m2
        coef_ref[...] = jnp.where(is1, m1, 0.0) + jnp.where(is2, m2, 0.0)
        out_ref[...] = jnp.zeros_like(out_ref)

    xb = x_ref[...].astype(jnp.bfloat16)
    w1 = w1_ref[0].astype(jnp.bfloat16)
    w2 = w2_ref[0].astype(jnp.bfloat16)
    h = jax.lax.dot_general(xb, w1, (((1,), (0,)), ((), ())),
                            preferred_element_type=jnp.float32)
    h = _gelu_tanh(h)
    o = jax.lax.dot_general(h.astype(jnp.bfloat16), w2,
                            (((1,), (0,)), ((), ())),
                            preferred_element_type=jnp.float32)
    onehot = (jax.lax.broadcasted_iota(jnp.int32, (E, 1), 0) == e
              ).astype(jnp.float32)
    ce = jax.lax.dot_general(coef_ref[...], onehot,
                             (((1,), (0,)), ((), ())),
                             preferred_element_type=jnp.float32)
    out_ref[...] += o * ce


@functools.partial(jax.jit, static_argnames=())
def kernel(x, ln_w, ln_b, gate_w, W1, W2):
    Bb, T, _ = x.shape
    x_flat = x.reshape(T, D)
    out = pl.pallas_call(
        _moe_body,
        grid=(E, NFF),
        in_specs=[
            pl.BlockSpec((T, D), lambda e, f: (0, 0)),
            pl.BlockSpec((D,), lambda e, f: (0,)),
            pl.BlockSpec((D,), lambda e, f: (0,)),
            pl.BlockSpec((E, D), lambda e, f: (0, 0)),
            pl.BlockSpec((1, D, FFC), lambda e, f: (e, 0, f)),
            pl.BlockSpec((1, FFC, D), lambda e, f: (e, f, 0)),
        ],
        out_specs=pl.BlockSpec((T, D), lambda e, f: (0, 0)),
        out_shape=jax.ShapeDtypeStruct((T, D), jnp.float32),
        scratch_shapes=[pltpu.VMEM((T, E), jnp.float32)],
        compiler_params=pltpu.CompilerParams(
            dimension_semantics=("arbitrary", "arbitrary")),
    )(x_flat, ln_w, ln_b, gate_w, W1, W2)
    return out.reshape(Bb, T, D)


# speedup vs baseline: 2.7757x; 1.0004x over previous
import functools

import jax
import jax.numpy as jnp
from jax.experimental import pallas as pl
from jax.experimental.pallas import tpu as pltpu

E = 8
TOP_K = 2
D = 1024
FF = 2048
FFC = 512
NFF = FF // FFC


def _gelu_tanh(h):
    c = 0.7978845608028654
    return 0.5 * h * (1.0 + jnp.tanh(c * (h + 0.044715 * h * h * h)))


def _moe_body(x_ref, lnw_ref, lnb_ref, gate_ref, w1_ref, w2_ref,
              out_ref, coef_ref, xb_ref):
    e = pl.program_id(0)
    f = pl.program_id(1)
    step = e * NFF + f

    @pl.when(step == 0)
    def _router():
        xf = x_ref[...]
        xb_ref[...] = xf.astype(jnp.bfloat16)
        mu = jnp.mean(xf, axis=1, keepdims=True)
        xc = xf - mu
        var = jnp.mean(xc * xc, axis=1, keepdims=True)
        ri = xc * jax.lax.rsqrt(var + 1e-5) * lnw_ref[...][None, :] \
            + lnb_ref[...][None, :]
        logits = jax.lax.dot_general(
            ri, gate_ref[...], (((1,), (1,)), ((), ())),
            preferred_element_type=jnp.float32)
        mx = jnp.max(logits, axis=1, keepdims=True)
        p = jnp.exp(logits - mx)
        p = p / jnp.sum(p, axis=1, keepdims=True)
        m1 = jnp.max(p, axis=1, keepdims=True)
        is1 = p >= m1
        p2 = jnp.where(is1, -jnp.inf, p)
        m2 = jnp.max(p2, axis=1, keepdims=True)
        is2 = p2 >= m2
        coef_ref[...] = jnp.where(is1, m1, 0.0) + jnp.where(is2, m2, 0.0)
        out_ref[...] = jnp.zeros_like(out_ref)

    xb = xb_ref[...]
    w1 = w1_ref[0].astype(jnp.bfloat16)
    w2 = w2_ref[0].astype(jnp.bfloat16)
    h = jax.lax.dot_general(xb, w1, (((1,), (0,)), ((), ())),
                            preferred_element_type=jnp.float32)
    h = _gelu_tanh(h)
    o = jax.lax.dot_general(h.astype(jnp.bfloat16), w2,
                            (((1,), (0,)), ((), ())),
                            preferred_element_type=jnp.float32)
    onehot = (jax.lax.broadcasted_iota(jnp.int32, (E, 1), 0) == e
              ).astype(jnp.float32)
    ce = jax.lax.dot_general(coef_ref[...], onehot,
                             (((1,), (0,)), ((), ())),
                             preferred_element_type=jnp.float32)
    out_ref[...] += o * ce


@functools.partial(jax.jit, static_argnames=())
def kernel(x, ln_w, ln_b, gate_w, W1, W2):
    Bb, T, _ = x.shape
    x_flat = x.reshape(T, D)
    out = pl.pallas_call(
        _moe_body,
        grid=(E, NFF),
        in_specs=[
            pl.BlockSpec((T, D), lambda e, f: (0, 0)),
            pl.BlockSpec((D,), lambda e, f: (0,)),
            pl.BlockSpec((D,), lambda e, f: (0,)),
            pl.BlockSpec((E, D), lambda e, f: (0, 0)),
            pl.BlockSpec((1, D, FFC), lambda e, f: (e, 0, f)),
            pl.BlockSpec((1, FFC, D), lambda e, f: (e, f, 0)),
        ],
        out_specs=pl.BlockSpec((T, D), lambda e, f: (0, 0)),
        out_shape=jax.ShapeDtypeStruct((T, D), jnp.float32),
        scratch_shapes=[pltpu.VMEM((T, E), jnp.float32),
                        pltpu.VMEM((T, D), jnp.bfloat16)],
        compiler_params=pltpu.CompilerParams(
            dimension_semantics=("arbitrary", "arbitrary")),
    )(x_flat, ln_w, ln_b, gate_w, W1, W2)
    return out.reshape(Bb, T, D)


# speedup vs baseline: 3.1120x; 1.1211x over previous
import functools

import jax
import jax.numpy as jnp
from jax import lax
from jax.experimental import pallas as pl
from jax.experimental.pallas import tpu as pltpu
from jax.experimental.pallas import tpu_sc as plsc

E = 8
TOP_K = 2
D = 1024
FF = 2048
T = 2048
A = T * TOP_K
BLK = 128
NB = A // BLK + E - 1
NB = 40
R = NB * BLK

NW = 32
TPW = T // NW
APW = A // NW


def _gelu_tanh(h):
    c = 0.7978845608028654
    return 0.5 * h * (1.0 + jnp.tanh(c * (h + 0.044715 * h * h * h)))


def _router_body(x_ref, lnw_ref, lnb_ref, gate_ref,
                 posw_ref, w0x_ref, w1x_ref, meta_ref):
    xf = x_ref[...]
    mu = jnp.mean(xf, axis=1, keepdims=True)
    xc = xf - mu
    var = jnp.mean(xc * xc, axis=1, keepdims=True)
    ri = xc * lax.rsqrt(var + 1e-5) * lnw_ref[...][None, :] \
        + lnb_ref[...][None, :]
    logits = lax.dot_general(ri, gate_ref[...], (((1,), (1,)), ((), ())),
                             preferred_element_type=jnp.float32)
    lt = logits.T
    mx = jnp.max(lt, axis=0, keepdims=True)
    pe = jnp.exp(lt - mx)
    p = pe / jnp.sum(pe, axis=0, keepdims=True)

    ei = lax.broadcasted_iota(jnp.int32, (E, E), 0)
    ej = lax.broadcasted_iota(jnp.int32, (E, E), 1)
    l8i = (ej <= ei).astype(jnp.float32)

    m1 = jnp.max(p, axis=0, keepdims=True)
    is1 = (p >= m1).astype(jnp.float32)
    cs1 = lax.dot_general(l8i, is1, (((1,), (0,)), ((), ())),
                          preferred_element_type=jnp.float32)
    is1 = is1 * (cs1 == 1.0).astype(jnp.float32)
    p2 = jnp.where(is1 == 1.0, -1.0, p)
    m2 = jnp.max(p2, axis=0, keepdims=True)
    is2 = (p2 >= m2).astype(jnp.float32)
    cs2 = lax.dot_general(l8i, is2, (((1,), (0,)), ((), ())),
                          preferred_element_type=jnp.float32)
    is2 = is2 * (cs2 == 1.0).astype(jnp.float32)

    ti = lax.broadcasted_iota(jnp.int32, (T, T), 0)
    tj = lax.broadcasted_iota(jnp.int32, (T, T), 1)
    u = (ti < tj).astype(jnp.float32)
    is12 = jnp.concatenate([is1, is2], axis=0)
    r12 = lax.dot_general(is12, u, (((1,), (0,)), ((), ())),
                          preferred_element_type=jnp.float32)
    r1 = r12[:E, :]
    r2 = r12[E:, :]

    c1 = jnp.sum(is1, axis=1, keepdims=True)
    c2 = jnp.sum(is2, axis=1, keepdims=True)
    cnt = c1 + c2
    pad = jnp.floor((cnt + (BLK - 1.0)) * (1.0 / BLK)) * BLK
    l8s = (ej < ei).astype(jnp.float32)
    base = lax.dot_general(l8s, pad, (((1,), (0,)), ((), ())),
                           preferred_element_type=jnp.float32)
    pos1 = jnp.sum(is1 * (base + r1), axis=0, keepdims=True)
    pos2 = jnp.sum(is2 * (base + c1 + r2), axis=0, keepdims=True)
    posw_ref[0:1, :] = pos1.astype(jnp.int32)
    posw_ref[1:2, :] = pos2.astype(jnp.int32)
    w0x_ref[...] = jnp.broadcast_to(m1.T, (T, 16))
    w1x_ref[...] = jnp.broadcast_to(m2.T, (T, 16))

    end = base + pad
    bvec = lax.broadcasted_iota(jnp.int32, (1, 128), 1).astype(jnp.float32) * BLK
    emap = jnp.sum((bvec >= end).astype(jnp.float32), axis=0, keepdims=True)
    emap = jnp.minimum(emap, float(E - 1))
    meta_ref[...] = jnp.broadcast_to(emap, (8, 128)).astype(jnp.int32)


def _router_call(x_flat, ln_w, ln_b, gate_w):
    return pl.pallas_call(
        _router_body,
        in_specs=[
            pl.BlockSpec((T, D), lambda: (0, 0)),
            pl.BlockSpec((D,), lambda: (0,)),
            pl.BlockSpec((D,), lambda: (0,)),
            pl.BlockSpec((E, D), lambda: (0, 0)),
        ],
        out_specs=[
            pl.BlockSpec((TOP_K, T), lambda: (0, 0)),
            pl.BlockSpec((T, 16), lambda: (0, 0)),
            pl.BlockSpec((T, 16), lambda: (0, 0)),
            pl.BlockSpec((8, 128), lambda: (0, 0)),
        ],
        out_shape=[
            jax.ShapeDtypeStruct((TOP_K, T), jnp.int32),
            jax.ShapeDtypeStruct((T, 16), jnp.float32),
            jax.ShapeDtypeStruct((T, 16), jnp.float32),
            jax.ShapeDtypeStruct((8, 128), jnp.int32),
        ],
    )(x_flat, ln_w, ln_b, gate_w)


_CH = 64


@functools.cache
def _make_dispatch_sc():
    mesh = plsc.VectorSubcoreMesh(core_axis_name="c", subcore_axis_name="s")

    @functools.partial(
        pl.kernel, mesh=mesh,
        out_type=jax.ShapeDtypeStruct((R, D), jnp.float32),
        scratch_types=[
            pltpu.VMEM((_CH,), jnp.int32),
            pltpu.VMEM((_CH, D), jnp.float32),
            pltpu.SemaphoreType.DMA,
        ],
    )
    def _dispatch_sc(x_hbm, posw_hbm, xs_hbm, pos_v, rows_v, sem):
        wid = lax.axis_index("s") * 2 + lax.axis_index("c")
        k = wid // 16
        tb = (wid % 16) * APW
        for c in range(APW // _CH):
            off = tb + c * _CH
            pltpu.sync_copy(posw_hbm.at[k, pl.ds(off, _CH)], pos_v)
            pltpu.sync_copy(x_hbm.at[pl.ds(off, _CH)], rows_v)
            pltpu.async_copy(rows_v, xs_hbm.at[pos_v], sem).wait()

    return _dispatch_sc


def _gemm_body(emap_ref, xs_ref, w1_ref, w2_ref, os_ref):
    xb = xs_ref[...].astype(jnp.bfloat16)
    w1 = w1_ref[0].astype(jnp.bfloat16)
    w2 = w2_ref[0].astype(jnp.bfloat16)
    h = lax.dot_general(xb, w1, (((1,), (0,)), ((), ())),
                        preferred_element_type=jnp.float32)
    h = _gelu_tanh(h).astype(jnp.bfloat16)
    os_ref[...] = lax.dot_general(h, w2, (((1,), (0,)), ((), ())),
                                  preferred_element_type=jnp.float32)


def _gemm_call(xs, W1, W2, emap):
    grid_spec = pltpu.PrefetchScalarGridSpec(
        num_scalar_prefetch=1,
        grid=(NB,),
        in_specs=[
            pl.BlockSpec((BLK, D), lambda b, emap: (b, 0)),
            pl.BlockSpec((1, D, FF), lambda b, emap: (emap[b], 0, 0)),
            pl.BlockSpec((1, FF, D), lambda b, emap: (emap[b], 0, 0)),
        ],
        out_specs=pl.BlockSpec((BLK, D), lambda b, emap: (b, 0)),
    )
    return pl.pallas_call(
        _gemm_body,
        grid_spec=grid_spec,
        out_shape=jax.ShapeDtypeStruct((R, D), jnp.float32),
        compiler_params=pltpu.CompilerParams(
            dimension_semantics=("arbitrary",)),
    )(emap, xs, W1, W2)


_CC = 32


@functools.cache
def _make_combine_sc():
    mesh = plsc.VectorSubcoreMesh(core_axis_name="c", subcore_axis_name="s")

    @functools.partial(
        pl.kernel, mesh=mesh,
        out_type=jax.ShapeDtypeStruct((T, D), jnp.float32),
        scratch_types=[
            pltpu.VMEM((_CC,), jnp.int32),
            pltpu.VMEM((_CC,), jnp.int32),
            pltpu.VMEM((_CC, 16), jnp.float32),
            pltpu.VMEM((_CC, 16), jnp.float32),
            pltpu.VMEM((_CC, D), jnp.float32),
            pltpu.VMEM((_CC, D), jnp.float32),
            pltpu.VMEM((_CC, D), jnp.float32),
            pltpu.SemaphoreType.DMA,
            pltpu.SemaphoreType.DMA,
        ],
    )
    def _combine_sc(os_hbm, posw_hbm, w0x_hbm, w1x_hbm, out_hbm,
                    p0_v, p1_v, w0_v, w1_v, a_v, b_v, o_v, sem0, sem1):
        wid = lax.axis_index("s") * 2 + lax.axis_index("c")
        tb = wid * TPW
        for c in range(TPW // _CC):
            off = tb + c * _CC
            pltpu.sync_copy(posw_hbm.at[0, pl.ds(off, _CC)], p0_v)
            pltpu.sync_copy(posw_hbm.at[1, pl.ds(off, _CC)], p1_v)
            pltpu.sync_copy(w0x_hbm.at[pl.ds(off, _CC)], w0_v)
            pltpu.sync_copy(w1x_hbm.at[pl.ds(off, _CC)], w1_v)
            cp0 = pltpu.async_copy(os_hbm.at[p0_v], a_v, sem0)
            cp1 = pltpu.async_copy(os_hbm.at[p1_v], b_v, sem1)
            cp0.wait()
            cp1.wait()

            def _row(i, carry):
                w0s = w0_v[i]
                w1s = w1_v[i]

                def _lane(j, carry2):
                    sl = pl.ds(j * 16, 16)
                    o_v[i, sl] = a_v[i, sl] * w0s + b_v[i, sl] * w1s
                    return carry2

                return lax.fori_loop(0, D // 16, _lane, carry)

            lax.fori_loop(0, _CC, _row, 0)
            pltpu.sync_copy(o_v, out_hbm.at[pl.ds(off, _CC)])

    return _combine_sc


@jax.jit
def kernel(x, ln_w, ln_b, gate_w, W1, W2):
    Bb = x.shape[0]
    x_flat = x.reshape(T, D)
    posw, w0x, w1x, meta = _router_call(x_flat, ln_w, ln_b, gate_w)
    emap = meta[0, :NB]
    xs = _make_dispatch_sc()(x_flat, posw)
    os_ = _gemm_call(xs, W1, W2, emap)
    out = _make_combine_sc()(os_, posw, w0x, w1x)
    return out.reshape(Bb, T, D)
